# baseline (speedup 1.0000x reference)
import functools

import jax
import jax.numpy as jnp
from jax import lax
from jax.experimental import pallas as pl
from jax.experimental.pallas import tpu as pltpu
from jax.experimental.pallas import tpu_sc as plsc

N = 10000
E = 320000
D = 128
G = 16

NTILES = 32
CPT = D // NTILES
FLAT = N * CPT
ECH = 8000
NCH = E // ECH
RB = 1000
NRB = N // RB


def _mm1_body(x_ref, w_ref, b_ref, u_ref, v_ref):
    x = x_ref[...]
    w = w_ref[...]
    wb = w[D:, :]
    v_ref[...] = jnp.dot(x, wb, preferred_element_type=jnp.float32)
    u_ref[...] = (
        jnp.dot(x, w[:D, :] - wb, preferred_element_type=jnp.float32)
        + b_ref[...]
    )


def _mm2_body(u1_ref, m1_ref, w_ref, b_ref, u_ref, v_ref):
    m = m1_ref[...]
    h = jnp.where(jnp.isfinite(m), u1_ref[...] + m, 0.0)
    w = w_ref[...]
    wb = w[D:, :]
    v_ref[...] = jnp.dot(h, wb, preferred_element_type=jnp.float32)
    u_ref[...] = (
        jnp.dot(h, w[:D, :] - wb, preferred_element_type=jnp.float32)
        + b_ref[...]
    )


def _pool_body(u_ref, m_ref, batch_ref, out_ref):
    i = pl.program_id(0)

    @pl.when(i == 0)
    def _():
        out_ref[...] = jnp.full((G, D), -jnp.inf, dtype=jnp.float32)

    m = m_ref[...]
    h2 = jnp.where(jnp.isfinite(m), u_ref[...] + m, 0.0)
    b = batch_ref[...]
    parts = jnp.concatenate(
        [
            jnp.max(jnp.where(b == g, h2, -jnp.inf), axis=0, keepdims=True)
            for g in range(G)
        ],
        axis=0,
    )
    out_ref[...] = jnp.maximum(out_ref[...], parts)

    @pl.when(i == NRB - 1)
    def _():
        o = out_ref[...]
        out_ref[...] = jnp.where(jnp.isfinite(o), o, 0.0)


def _mm1(x, w, b):
    return pl.pallas_call(
        _mm1_body,
        grid=(NRB,),
        in_specs=[
            pl.BlockSpec((RB, D), lambda i: (i, 0)),
            pl.BlockSpec((2 * D, D), lambda i: (0, 0)),
            pl.BlockSpec((1, D), lambda i: (0, 0)),
        ],
        out_specs=[
            pl.BlockSpec((RB, D), lambda i: (i, 0)),
            pl.BlockSpec((RB, D), lambda i: (i, 0)),
        ],
        out_shape=[
            jax.ShapeDtypeStruct((N, D), jnp.float32),
            jax.ShapeDtypeStruct((N, D), jnp.float32),
        ],
    )(x, w, b)


def _mm2(u1, m1, w, b):
    return pl.pallas_call(
        _mm2_body,
        grid=(NRB,),
        in_specs=[
            pl.BlockSpec((RB, D), lambda i: (i, 0)),
            pl.BlockSpec((RB, D), lambda i: (i, 0)),
            pl.BlockSpec((2 * D, D), lambda i: (0, 0)),
            pl.BlockSpec((1, D), lambda i: (0, 0)),
        ],
        out_specs=[
            pl.BlockSpec((RB, D), lambda i: (i, 0)),
            pl.BlockSpec((RB, D), lambda i: (i, 0)),
        ],
        out_shape=[
            jax.ShapeDtypeStruct((N, D), jnp.float32),
            jax.ShapeDtypeStruct((N, D), jnp.float32),
        ],
    )(u1, m1, w, b)


def _pool(u2, m2, batch2d):
    return pl.pallas_call(
        _pool_body,
        grid=(NRB,),
        in_specs=[
            pl.BlockSpec((RB, D), lambda i: (i, 0)),
            pl.BlockSpec((RB, D), lambda i: (i, 0)),
            pl.BlockSpec((RB, 1), lambda i: (i, 0)),
        ],
        out_specs=pl.BlockSpec((G, D), lambda i: (0, 0)),
        out_shape=jax.ShapeDtypeStruct((G, D), jnp.float32),
    )(u2, m2, batch2d)


def _segmax(vb, src, dst):
    mesh = plsc.VectorSubcoreMesh(core_axis_name="c", subcore_axis_name="s")

    @functools.partial(
        pl.kernel,
        out_type=jax.ShapeDtypeStruct((NTILES, FLAT), jnp.float32),
        mesh=mesh,
        compiler_params=pltpu.CompilerParams(needs_layout_passes=False),
        scratch_types=[
            pltpu.VMEM((FLAT,), jnp.float32),
            pltpu.VMEM((FLAT,), jnp.float32),
            pltpu.VMEM((ECH,), jnp.int32),
            pltpu.VMEM((ECH,), jnp.int32),
        ],
    )
    def k(vb_hbm, src_hbm, dst_hbm, out_hbm, vloc, mloc, sbuf, dbuf):
        wid = lax.axis_index("c") * 16 + lax.axis_index("s")
        pltpu.sync_copy(vb_hbm.at[wid], vloc)

        neg_inf = jnp.broadcast_to(jnp.float32(-jnp.inf), (16,))

        def init_body(i, carry):
            mloc[pl.ds(i * 16, 16)] = neg_inf
            return carry

        lax.fori_loop(0, FLAT // 16, init_body, 0)

        lanes = lax.iota(jnp.int32, 16)
        sh = [jnp.maximum(lanes - k, 0) for k in (1, 2, 4, 8)]
        ge = [lanes >= k for k in (1, 2, 4, 8)]
        nxt = jnp.minimum(lanes + 1, 15)
        is15 = lanes == 15

        def chunk_body(ci, carry):
            pltpu.sync_copy(src_hbm.at[pl.ds(ci * ECH, ECH)], sbuf)
            pltpu.sync_copy(dst_hbm.at[pl.ds(ci * ECH, ECH)], dbuf)

            def group_body(g, c2):
                s = sbuf[pl.ds(g * 16, 16)]
                d = dbuf[pl.ds(g * 16, 16)]
                d_s, s_s = plsc.sort_key_val(d, s)
                dsh = [
                    jnp.take_along_axis(d_s, ix, axis=0) for ix in sh
                ]
                runm = [(d_s == dv) & g_ for dv, g_ in zip(dsh, ge)]
                last = (
                    d_s != jnp.take_along_axis(d_s, nxt, axis=0)
                ) | is15
                vb4 = s_s * CPT
                mb4 = d_s * CPT
                for c in range(CPT):
                    v = plsc.load_gather(vloc, [vb4 + c])
                    for ix, rm in zip(sh, runm):
                        vsh = jnp.take_along_axis(v, ix, axis=0)
                        v = jnp.where(rm, jnp.maximum(v, vsh), v)
                    m = plsc.load_gather(mloc, [mb4 + c], mask=last)
                    plsc.store_scatter(
                        mloc, [mb4 + c], jnp.maximum(v, m), mask=last
                    )
                return c2

            lax.fori_loop(0, ECH // 16, group_body, 0)
            return carry

        lax.fori_loop(0, NCH, chunk_body, 0)
        pltpu.sync_copy(mloc, out_hbm.at[wid])

    return k(vb, src, dst)


def _to_blocked(v):
    return v.reshape(N, NTILES, CPT).transpose(1, 0, 2).reshape(NTILES, FLAT)


def _from_blocked(mb):
    return mb.reshape(NTILES, N, CPT).transpose(1, 0, 2).reshape(N, D)


def kernel(x, edge_index, batch, W1, b1, W2, b2):
    src = edge_index[0]
    dst = edge_index[1]
    b1r = b1.reshape(1, D)
    b2r = b2.reshape(1, D)

    u1, v1 = _mm1(x, W1, b1r)
    m1 = _from_blocked(_segmax(_to_blocked(v1), src, dst))
    u2, v2 = _mm2(u1, m1, W2, b2r)
    m2 = _from_blocked(_segmax(_to_blocked(v2), src, dst))
    return _pool(u2, m2, batch.reshape(N, 1))


# speedup vs baseline: 1.1142x; 1.1142x over previous
import functools

import jax
import jax.numpy as jnp
from jax import lax
from jax.experimental import pallas as pl
from jax.experimental.pallas import tpu as pltpu
from jax.experimental.pallas import tpu_sc as plsc

N = 10000
E = 320000
D = 128
G = 16

NTILES = 32
CPT = D // NTILES
FLAT = N * CPT
ECH = 8000
NCH = E // ECH
RB = 1000
NRB = N // RB


def _mm1_body(x_ref, w_ref, b_ref, u_ref, v_ref):
    x = x_ref[...]
    w = w_ref[...]
    wb = w[D:, :]
    v_ref[...] = jnp.dot(x, wb, preferred_element_type=jnp.float32)
    u_ref[...] = (
        jnp.dot(x, w[:D, :] - wb, preferred_element_type=jnp.float32)
        + b_ref[...]
    )


def _mm2_body(u1_ref, m1_ref, w_ref, b_ref, u_ref, v_ref):
    m = m1_ref[...]
    h = jnp.where(jnp.isfinite(m), u1_ref[...] + m, 0.0)
    w = w_ref[...]
    wb = w[D:, :]
    v_ref[...] = jnp.dot(h, wb, preferred_element_type=jnp.float32)
    u_ref[...] = (
        jnp.dot(h, w[:D, :] - wb, preferred_element_type=jnp.float32)
        + b_ref[...]
    )


def _pool_body(u_ref, m_ref, batch_ref, out_ref):
    i = pl.program_id(0)

    @pl.when(i == 0)
    def _():
        out_ref[...] = jnp.full((G, D), -jnp.inf, dtype=jnp.float32)

    m = m_ref[...]
    h2 = jnp.where(jnp.isfinite(m), u_ref[...] + m, 0.0)
    b = batch_ref[...]
    parts = jnp.concatenate(
        [
            jnp.max(jnp.where(b == g, h2, -jnp.inf), axis=0, keepdims=True)
            for g in range(G)
        ],
        axis=0,
    )
    out_ref[...] = jnp.maximum(out_ref[...], parts)

    @pl.when(i == NRB - 1)
    def _():
        o = out_ref[...]
        out_ref[...] = jnp.where(jnp.isfinite(o), o, 0.0)


def _mm1(x, w, b):
    return pl.pallas_call(
        _mm1_body,
        grid=(NRB,),
        in_specs=[
            pl.BlockSpec((RB, D), lambda i: (i, 0)),
            pl.BlockSpec((2 * D, D), lambda i: (0, 0)),
            pl.BlockSpec((1, D), lambda i: (0, 0)),
        ],
        out_specs=[
            pl.BlockSpec((RB, D), lambda i: (i, 0)),
            pl.BlockSpec((RB, D), lambda i: (i, 0)),
        ],
        out_shape=[
            jax.ShapeDtypeStruct((N, D), jnp.float32),
            jax.ShapeDtypeStruct((N, D), jnp.float32),
        ],
    )(x, w, b)


def _mm2(u1, m1, w, b):
    return pl.pallas_call(
        _mm2_body,
        grid=(NRB,),
        in_specs=[
            pl.BlockSpec((RB, D), lambda i: (i, 0)),
            pl.BlockSpec((RB, D), lambda i: (i, 0)),
            pl.BlockSpec((2 * D, D), lambda i: (0, 0)),
            pl.BlockSpec((1, D), lambda i: (0, 0)),
        ],
        out_specs=[
            pl.BlockSpec((RB, D), lambda i: (i, 0)),
            pl.BlockSpec((RB, D), lambda i: (i, 0)),
        ],
        out_shape=[
            jax.ShapeDtypeStruct((N, D), jnp.float32),
            jax.ShapeDtypeStruct((N, D), jnp.float32),
        ],
    )(u1, m1, w, b)


def _pool(u2, m2, batch2d):
    return pl.pallas_call(
        _pool_body,
        grid=(NRB,),
        in_specs=[
            pl.BlockSpec((RB, D), lambda i: (i, 0)),
            pl.BlockSpec((RB, D), lambda i: (i, 0)),
            pl.BlockSpec((RB, 1), lambda i: (i, 0)),
        ],
        out_specs=pl.BlockSpec((G, D), lambda i: (0, 0)),
        out_shape=jax.ShapeDtypeStruct((G, D), jnp.float32),
    )(u2, m2, batch2d)


def _segmax(vb, src, dst):
    mesh = plsc.VectorSubcoreMesh(core_axis_name="c", subcore_axis_name="s")

    @functools.partial(
        pl.kernel,
        out_type=jax.ShapeDtypeStruct((NTILES, CPT, N), jnp.float32),
        mesh=mesh,
        compiler_params=pltpu.CompilerParams(needs_layout_passes=False),
        scratch_types=[
            [pltpu.VMEM((N,), jnp.float32) for _ in range(CPT)],
            [pltpu.VMEM((N,), jnp.float32) for _ in range(CPT)],
            pltpu.VMEM((ECH,), jnp.int32),
            pltpu.VMEM((ECH,), jnp.int32),
        ],
    )
    def k(vb_hbm, src_hbm, dst_hbm, out_hbm, vlocs, mlocs, sbuf, dbuf):
        wid = lax.axis_index("c") * 16 + lax.axis_index("s")
        for c in range(CPT):
            pltpu.sync_copy(vb_hbm.at[wid, c], vlocs[c])

        neg_inf = jnp.broadcast_to(jnp.float32(-jnp.inf), (16,))

        def init_body(i, carry):
            for c in range(CPT):
                mlocs[c][pl.ds(i * 16, 16)] = neg_inf
            return carry

        lax.fori_loop(0, N // 16, init_body, 0)

        lanes = lax.iota(jnp.int32, 16)
        sh = [jnp.maximum(lanes - k, 0) for k in (1, 2, 4, 8)]
        ge = [lanes >= k for k in (1, 2, 4, 8)]
        nxt = jnp.minimum(lanes + 1, 15)
        is15 = lanes == 15

        def one_group(g):
            s = sbuf[pl.ds(g * 16, 16)]
            d = dbuf[pl.ds(g * 16, 16)]
            d_s, s_s = plsc.sort_key_val(d, s)
            dsh = [jnp.take_along_axis(d_s, ix, axis=0) for ix in sh]
            runm = [(d_s == dv) & g_ for dv, g_ in zip(dsh, ge)]
            last = (d_s != jnp.take_along_axis(d_s, nxt, axis=0)) | is15
            for c in range(CPT):
                v = plsc.load_gather(vlocs[c], [s_s])
                for ix, rm in zip(sh, runm):
                    vsh = jnp.take_along_axis(v, ix, axis=0)
                    v = jnp.where(rm, jnp.maximum(v, vsh), v)
                m = plsc.load_gather(mlocs[c], [d_s], mask=last)
                plsc.store_scatter(
                    mlocs[c], [d_s], jnp.maximum(v, m), mask=last
                )

        def chunk_body(ci, carry):
            pltpu.sync_copy(src_hbm.at[pl.ds(ci * ECH, ECH)], sbuf)
            pltpu.sync_copy(dst_hbm.at[pl.ds(ci * ECH, ECH)], dbuf)

            def group_body(g2, c2):
                one_group(g2 * 2)
                one_group(g2 * 2 + 1)
                return c2

            lax.fori_loop(0, ECH // 32, group_body, 0)
            return carry

        lax.fori_loop(0, NCH, chunk_body, 0)
        for c in range(CPT):
            pltpu.sync_copy(mlocs[c], out_hbm.at[wid, c])

    return k(vb, src, dst)


def _to_blocked(v):
    return v.T.reshape(NTILES, CPT, N)


def _from_blocked(mb):
    return mb.reshape(D, N).T


def kernel(x, edge_index, batch, W1, b1, W2, b2):
    src = edge_index[0]
    dst = edge_index[1]
    b1r = b1.reshape(1, D)
    b2r = b2.reshape(1, D)

    u1, v1 = _mm1(x, W1, b1r)
    m1 = _from_blocked(_segmax(_to_blocked(v1), src, dst))
    u2, v2 = _mm2(u1, m1, W2, b2r)
    m2 = _from_blocked(_segmax(_to_blocked(v2), src, dst))
    return _pool(u2, m2, batch.reshape(N, 1))


# speedup vs baseline: 6.3141x; 5.6670x over previous
import functools

import jax
import jax.numpy as jnp
from jax import lax
from jax.experimental import pallas as pl
from jax.experimental.pallas import tpu as pltpu
from jax.experimental.pallas import tpu_sc as plsc

N = 10000
E = 320000
D = 128
G = 16

NTILES = 32
CPT = D // NTILES
FLAT = N * CPT
ECH = 8000
NCH = E // ECH
NPAD = 10240
RB = 1024
NRB = NPAD // RB


def _mm1_body(xt_ref, wt_ref, b_ref, ut_ref, vt_ref):
    xt = xt_ref[...]
    wt = wt_ref[...]
    wbt = wt[:, D:]
    vt_ref[...] = jnp.dot(wbt, xt, preferred_element_type=jnp.float32)
    ut_ref[...] = (
        jnp.dot(wt[:, :D] - wbt, xt, preferred_element_type=jnp.float32)
        + b_ref[...]
    )


def _mm2_body(ut1_ref, mt1_ref, wt_ref, b_ref, ut_ref, vt_ref):
    mt = mt1_ref[...]
    ht = jnp.where(jnp.isfinite(mt), ut1_ref[...] + mt, 0.0)
    wt = wt_ref[...]
    wbt = wt[:, D:]
    vt_ref[...] = jnp.dot(wbt, ht, preferred_element_type=jnp.float32)
    ut_ref[...] = (
        jnp.dot(wt[:, :D] - wbt, ht, preferred_element_type=jnp.float32)
        + b_ref[...]
    )


def _pool_body(ut_ref, mt_ref, batch_ref, out_ref):
    i = pl.program_id(0)

    @pl.when(i == 0)
    def _():
        out_ref[...] = jnp.full((D, G), -jnp.inf, dtype=jnp.float32)

    mt = mt_ref[...]
    h2t = jnp.where(jnp.isfinite(mt), ut_ref[...] + mt, 0.0)
    b = batch_ref[...]
    parts = jnp.concatenate(
        [
            jnp.max(jnp.where(b == g, h2t, -jnp.inf), axis=1, keepdims=True)
            for g in range(G)
        ],
        axis=1,
    )
    out_ref[...] = jnp.maximum(out_ref[...], parts)

    @pl.when(i == NRB - 1)
    def _():
        o = out_ref[...]
        out_ref[...] = jnp.where(jnp.isfinite(o), o, 0.0)


def _mm1(xt, wt, b):
    return pl.pallas_call(
        _mm1_body,
        grid=(NRB,),
        in_specs=[
            pl.BlockSpec((D, RB), lambda i: (0, i)),
            pl.BlockSpec((D, 2 * D), lambda i: (0, 0)),
            pl.BlockSpec((D, 1), lambda i: (0, 0)),
        ],
        out_specs=[
            pl.BlockSpec((D, RB), lambda i: (0, i)),
            pl.BlockSpec((D, RB), lambda i: (0, i)),
        ],
        out_shape=[
            jax.ShapeDtypeStruct((D, NPAD), jnp.float32),
            jax.ShapeDtypeStruct((D, NPAD), jnp.float32),
        ],
    )(xt, wt, b)


def _mm2(ut1, mt1, wt, b):
    return pl.pallas_call(
        _mm2_body,
        grid=(NRB,),
        in_specs=[
            pl.BlockSpec((D, RB), lambda i: (0, i)),
            pl.BlockSpec((D, RB), lambda i: (0, i)),
            pl.BlockSpec((D, 2 * D), lambda i: (0, 0)),
            pl.BlockSpec((D, 1), lambda i: (0, 0)),
        ],
        out_specs=[
            pl.BlockSpec((D, RB), lambda i: (0, i)),
            pl.BlockSpec((D, RB), lambda i: (0, i)),
        ],
        out_shape=[
            jax.ShapeDtypeStruct((D, NPAD), jnp.float32),
            jax.ShapeDtypeStruct((D, NPAD), jnp.float32),
        ],
    )(ut1, mt1, wt, b)


def _pool(ut2, mt2, batch2d):
    return pl.pallas_call(
        _pool_body,
        grid=(NRB,),
        in_specs=[
            pl.BlockSpec((D, RB), lambda i: (0, i)),
            pl.BlockSpec((D, RB), lambda i: (0, i)),
            pl.BlockSpec((1, RB), lambda i: (0, i)),
        ],
        out_specs=pl.BlockSpec((D, G), lambda i: (0, 0)),
        out_shape=jax.ShapeDtypeStruct((D, G), jnp.float32),
    )(ut2, mt2, batch2d)


NPAIR = CPT // 2


def _segmax(vb, src, dst):
    mesh = plsc.VectorSubcoreMesh(core_axis_name="c", subcore_axis_name="s")

    @functools.partial(
        pl.kernel,
        out_type=jax.ShapeDtypeStruct((NTILES, NPAIR, NPAD), jnp.int32),
        mesh=mesh,
        compiler_params=pltpu.CompilerParams(needs_layout_passes=False),
        scratch_types=[
            [pltpu.VMEM((NPAD,), jnp.int32) for _ in range(NPAIR)],
            [pltpu.VMEM((NPAD,), jnp.int32) for _ in range(NPAIR)],
            [pltpu.VMEM((ECH,), jnp.int32) for _ in range(2)],
            [pltpu.VMEM((ECH,), jnp.int32) for _ in range(2)],
            [pltpu.SemaphoreType.DMA for _ in range(4)],
        ],
    )
    def k(vb_hbm, src_hbm, dst_hbm, out_hbm, vlocs, mlocs, sbufs, dbufs,
          sems):
        wid = lax.axis_index("c") * 16 + lax.axis_index("s")
        for c in range(NPAIR):
            pltpu.sync_copy(vb_hbm.at[wid, c], vlocs[c])

        neg_inf = jnp.broadcast_to(jnp.int32(-8323200), (16,))

        def init_body(i, carry):
            for c in range(NPAIR):
                mlocs[c][pl.ds(i * 16, 16)] = neg_inf
            return carry

        lax.fori_loop(0, NPAD // 16, init_body, 0)

        lanes = lax.iota(jnp.int32, 16)
        sh = [jnp.maximum(lanes - k, 0) for k in (1, 2, 4, 8)]
        ge = [lanes >= k for k in (1, 2, 4, 8)]
        nxt = jnp.minimum(lanes + 1, 15)
        is15 = lanes == 15

        UNROLL = 5

        def _start(ci, slot):
            ix = pl.ds(ci * ECH, ECH)
            pltpu.async_copy(src_hbm.at[ix], sbufs[slot], sems[2 * slot])
            pltpu.async_copy(dst_hbm.at[ix], dbufs[slot], sems[2 * slot + 1])

        def _wait(ci, slot):
            ix = pl.ds(ci * ECH, ECH)
            pltpu.make_async_copy(
                src_hbm.at[ix], sbufs[slot], sems[2 * slot]
            ).wait()
            pltpu.make_async_copy(
                dst_hbm.at[ix], dbufs[slot], sems[2 * slot + 1]
            ).wait()

        def make_group_body(nsteps, with_flags, slot):
            sref = sbufs[slot]
            dref = dbufs[slot]

            def group_body(gu, flags):
                gs = [gu * UNROLL + u for u in range(UNROLL)]
                sv = [sref[pl.ds(g * 16, 16)] for g in gs]
                dv = [dref[pl.ds(g * 16, 16)] for g in gs]
                srt = [plsc.sort_key_val(d, s) for d, s in zip(dv, sv)]
                d_s = [t[0] for t in srt]
                s_s = [t[1] for t in srt]
                runm = []
                lastm = []
                for u in range(UNROLL):
                    dsh = [
                        jnp.take_along_axis(d_s[u], ix, axis=0)
                        for ix in sh[:nsteps]
                    ]
                    runm.append(
                        [(d_s[u] == x) & g_ for x, g_ in zip(dsh, ge)]
                    )
                    lastm.append(
                        (d_s[u] != jnp.take_along_axis(d_s[u], nxt, axis=0))
                        | is15
                    )
                    if with_flags:
                        run3 = (
                            d_s[u]
                            == jnp.take_along_axis(d_s[u], sh[1], axis=0)
                        ) & ge[1]
                        flags = flags | run3
                runm32 = [
                    [
                        plsc.pack(
                            jnp.where(rm, 1, 0),
                            jnp.where(rm, 1, 0),
                            format=plsc.PackFormat.INTERLEAVED,
                        )
                        != 0
                        for rm in runm[u]
                    ]
                    for u in range(UNROLL)
                ]
                vals = [
                    [
                        plsc.load_gather(vlocs[c], [s_s[u]])
                        for c in range(NPAIR)
                    ]
                    for u in range(UNROLL)
                ]
                valsb = [
                    [plsc.bitcast(v, jnp.bfloat16) for v in vals[u]]
                    for u in range(UNROLL)
                ]
                for step, ix in enumerate(sh[:nsteps]):
                    for u in range(UNROLL):
                        rm32 = runm32[u][step]
                        for c in range(NPAIR):
                            v32 = plsc.bitcast(valsb[u][c], jnp.int32)
                            vsh = plsc.bitcast(
                                jnp.take_along_axis(v32, ix, axis=0),
                                jnp.bfloat16,
                            )
                            valsb[u][c] = jnp.where(
                                rm32,
                                jnp.maximum(valsb[u][c], vsh),
                                valsb[u][c],
                            )
                for u in range(UNROLL):
                    msb = [
                        plsc.bitcast(
                            plsc.load_gather(
                                mlocs[c], [d_s[u]], mask=lastm[u]
                            ),
                            jnp.bfloat16,
                        )
                        for c in range(NPAIR)
                    ]
                    for c in range(NPAIR):
                        plsc.store_scatter(
                            mlocs[c],
                            [d_s[u]],
                            plsc.bitcast(
                                jnp.maximum(valsb[u][c], msb[c]), jnp.int32
                            ),
                            mask=lastm[u],
                        )
                return flags

            return group_body

        nbatch = ECH // (16 * UNROLL)
        falses = lanes < 0
        fast_bodies = [make_group_body(1, True, s) for s in range(2)]
        full_bodies = [make_group_body(4, False, s) for s in range(2)]

        def _process(slot):
            flags = lax.fori_loop(0, nbatch, fast_bodies[slot], falses)

            def redo():
                lax.fori_loop(0, nbatch, full_bodies[slot], falses)

            lax.cond(jnp.any(flags), redo, lambda: None)

        _start(0, 0)

        def pair_body(cp, carry):
            ci0 = cp * 2
            _wait(ci0, 0)
            _start(ci0 + 1, 1)
            _process(0)
            _wait(ci0 + 1, 1)

            @pl.when(cp < NCH // 2 - 1)
            def _():
                _start(ci0 + 2, 0)

            _process(1)
            return carry

        lax.fori_loop(0, NCH // 2, pair_body, 0)
        for c in range(NPAIR):
            pltpu.sync_copy(mlocs[c], out_hbm.at[wid, c])

    return k(vb, src, dst)


def _to_blocked(vt):
    vtb = vt.astype(jnp.bfloat16).reshape(NTILES, NPAIR, 2, NPAD)
    return lax.bitcast_convert_type(
        vtb.transpose(0, 1, 3, 2), jnp.int32
    )


def _from_blocked(mb):
    mbb = lax.bitcast_convert_type(mb, jnp.bfloat16)
    return (
        mbb.transpose(0, 1, 3, 2).reshape(D, NPAD).astype(jnp.float32)
    )


def kernel(x, edge_index, batch, W1, b1, W2, b2):
    src = edge_index[0]
    dst = edge_index[1]
    xt = jnp.pad(x.T, ((0, 0), (0, NPAD - N)))
    batch2d = jnp.pad(batch, (0, NPAD - N), constant_values=G).reshape(
        1, NPAD
    )
    b1r = b1.reshape(D, 1)
    b2r = b2.reshape(D, 1)

    ut1, vt1 = _mm1(xt, W1.T, b1r)
    mt1 = _from_blocked(_segmax(_to_blocked(vt1), src, dst))
    ut2, vt2 = _mm2(ut1, mt1, W2.T, b2r)
    mt2 = _from_blocked(_segmax(_to_blocked(vt2), src, dst))
    return _pool(ut2, mt2, batch2d).T
